# Initial kernel scaffold; baseline (speedup 1.0000x reference)
import jax
import jax.numpy as jnp
from jax.experimental import pallas as pl


def kernel(x_nchw, w_oihw, bias):
    raise NotImplementedError("write your pallas kernel here")


# speedup vs baseline: 3.4312x; 3.4312x over previous
import jax
import jax.numpy as jnp
from jax.experimental import pallas as pl
from jax.experimental.pallas import tpu as pltpu


def _conv3x3s2_kernel(x_ref, w_ref, b_ref, o_ref):
    _, Hh, _, Wh, _, C = x_ref.shape
    Ho, Wo = Hh - 1, Wh - 1
    x = x_ref[0]
    acc = jnp.zeros((Ho * Wo, o_ref.shape[-1]), jnp.float32)
    for ky in range(3):
        h0, r = ky // 2, ky % 2
        for kx in range(3):
            w0, p = kx // 2, kx % 2
            lhs = x[h0:h0 + Ho, r, w0:w0 + Wo, p, :].reshape(Ho * Wo, C)
            acc = acc + jnp.dot(lhs, w_ref[ky, kx],
                                preferred_element_type=jnp.float32)
    o_ref[0] = acc + b_ref[...]


def kernel(x_nchw, w_oihw, bias):
    N, C, H, W = x_nchw.shape
    Cout = w_oihw.shape[0]
    Ho = (H - 2) // 2 + 1
    Wo = (W - 2) // 2 + 1
    Hp, Wp = 2 * Ho + 2, 2 * Wo + 2
    Hh, Wh = Hp // 2, Wp // 2

    x = jnp.transpose(x_nchw, (0, 2, 3, 1)).astype(jnp.bfloat16)
    x = jnp.pad(x, ((0, 0), (0, Hp - H), (0, Wp - W), (0, 0)))
    x = x.reshape(N, Hh, 2, Wh, 2, C)
    wt = jnp.transpose(w_oihw, (2, 3, 1, 0)).astype(jnp.bfloat16)
    b2 = bias.reshape(1, Cout).astype(jnp.float32)

    out = pl.pallas_call(
        _conv3x3s2_kernel,
        out_shape=jax.ShapeDtypeStruct((N, Ho * Wo, Cout), jnp.float32),
        grid=(N,),
        in_specs=[
            pl.BlockSpec((1, Hh, 2, Wh, 2, C), lambda n: (n, 0, 0, 0, 0, 0)),
            pl.BlockSpec((3, 3, C, Cout), lambda n: (0, 0, 0, 0)),
            pl.BlockSpec((1, Cout), lambda n: (0, 0)),
        ],
        out_specs=pl.BlockSpec((1, Ho * Wo, Cout), lambda n: (n, 0, 0)),
        compiler_params=pltpu.CompilerParams(
            dimension_semantics=("parallel",),
            vmem_limit_bytes=64 * 1024 * 1024),
    )(x, wt, b2)

    out = out.reshape(N, Ho, Wo, Cout)
    return jnp.transpose(out, (0, 3, 1, 2))
